# baseline (speedup 1.0000x reference)
import functools

import jax
import jax.numpy as jnp
from jax import lax
from jax.experimental import pallas as pl
from jax.experimental.pallas import tpu as pltpu
from jax.experimental.pallas import tpu_sc as plsc

N = 10000
D = 128
E = 320000

NC = 2
NS = 16
NW = NC * NS

C = 120
CHUNKS = 84
EPW = C * CHUNKS
E2 = EPW * NW
PAD = E2 - E
NA = N + 8

ROWS_A = 632
ROWS_B = N - 15 * ROWS_A

_mesh = plsc.VectorSubcoreMesh(core_axis_name="c", subcore_axis_name="s")


def _f32(*shape):
    return jax.ShapeDtypeStruct(shape, jnp.float32)


def _zero_acc(sid, zeros_hbm, acc):
    @pl.when(sid < NS - 1)
    def _():
        pltpu.sync_copy(zeros_hbm, acc.at[pl.ds(sid * ROWS_A, ROWS_A)])

    @pl.when(sid == NS - 1)
    def _():
        pltpu.sync_copy(zeros_hbm.at[pl.ds(0, ROWS_B)],
                        acc.at[pl.ds(sid * ROWS_A, ROWS_B)])


def _writeback(cid, sid, acc, p0_hbm, p1_hbm):
    @pl.when(sid < NS - 1)
    def _():
        @pl.when(cid == 0)
        def _():
            pltpu.sync_copy(acc.at[pl.ds(sid * ROWS_A, ROWS_A)],
                            p0_hbm.at[pl.ds(sid * ROWS_A, ROWS_A)])

        @pl.when(cid == 1)
        def _():
            pltpu.sync_copy(acc.at[pl.ds(sid * ROWS_A, ROWS_A)],
                            p1_hbm.at[pl.ds(sid * ROWS_A, ROWS_A)])

    @pl.when(sid == NS - 1)
    def _():
        @pl.when(cid == 0)
        def _():
            pltpu.sync_copy(acc.at[pl.ds(sid * ROWS_A, ROWS_B)],
                            p0_hbm.at[pl.ds(sid * ROWS_A, ROWS_B)])

        @pl.when(cid == 1)
        def _():
            pltpu.sync_copy(acc.at[pl.ds(sid * ROWS_A, ROWS_B)],
                            p1_hbm.at[pl.ds(sid * ROWS_A, ROWS_B)])


@functools.partial(
    pl.kernel,
    out_type=(_f32(N, D), _f32(N, D)),
    mesh=_mesh,
    scratch_types=[
        [pltpu.VMEM((C,), jnp.int32)] * 6,
        pltpu.VMEM((C, D), jnp.float32),
        pltpu.VMEM_SHARED((NA, D), jnp.float32),
        [pltpu.SemaphoreType.DMA] * 6,
        [pltpu.SemaphoreType.DMA] * 3,
    ],
)
def _degree_kernel(dst_hbm, ones_hbm, zeros_hbm, d0_hbm, d1_hbm,
                   dvs, ones_v, acc, lis, sss):
    cid = lax.axis_index("c")
    sid = lax.axis_index("s")
    wid = cid * NS + sid
    base = wid * EPW

    pltpu.sync_copy(ones_hbm, ones_v)
    _zero_acc(sid, zeros_hbm, acc)

    def idx_src(c):
        return dst_hbm.at[pl.ds(base + c * C, C)]

    for q in range(6):
        pltpu.async_copy(idx_src(q), dvs[q], lis[q])

    plsc.subcore_barrier()

    @pl.loop(0, CHUNKS, step=6)
    def _(j):
        for half in range(2):
            for p in range(3):
                q = half * 3 + p
                pltpu.make_async_copy(idx_src(0), dvs[q], lis[q]).wait()
                pltpu.async_copy(ones_v, acc.at[dvs[q]], sss[p], add=True)
            for p in range(3):
                q = half * 3 + p
                c2 = jnp.minimum(j + q + 6, CHUNKS - 1)
                pltpu.make_async_copy(ones_v, acc.at[dvs[q]], sss[p]).wait()
                pltpu.async_copy(idx_src(c2), dvs[q], lis[q])

    for q in range(6):
        pltpu.make_async_copy(idx_src(0), dvs[q], lis[q]).wait()

    plsc.subcore_barrier()
    _writeback(cid, sid, acc, d0_hbm, d1_hbm)


@functools.partial(
    pl.kernel,
    out_type=(_f32(N, D), _f32(N, D)),
    mesh=_mesh,
    scratch_types=[
        [pltpu.VMEM((C,), jnp.int32)] * 6,
        [pltpu.VMEM((C,), jnp.int32)] * 6,
        [pltpu.VMEM((C, D), jnp.float32)] * 3,
        pltpu.VMEM_SHARED((NA, D), jnp.float32),
        [pltpu.SemaphoreType.DMA] * 6,
        [pltpu.SemaphoreType.DMA] * 3,
        [pltpu.SemaphoreType.DMA] * 3,
    ],
)
def _propagate_kernel(g_hbm, src_hbm, dst_hbm, zeros_hbm, p0_hbm, p1_hbm,
                      svs, dvs, rws, acc, lis, ggs, sss):
    cid = lax.axis_index("c")
    sid = lax.axis_index("s")
    wid = cid * NS + sid
    base = wid * EPW

    def sidx(c):
        return src_hbm.at[pl.ds(base + c * C, C)]

    def didx(c):
        return dst_hbm.at[pl.ds(base + c * C, C)]

    _zero_acc(sid, zeros_hbm, acc)

    for q in range(6):
        pltpu.async_copy(sidx(q), svs[q], lis[q])
        pltpu.async_copy(didx(q), dvs[q], lis[q])

    plsc.subcore_barrier()

    for p in range(3):
        pltpu.make_async_copy(sidx(p), svs[p], lis[p]).wait()
        pltpu.make_async_copy(didx(p), dvs[p], lis[p]).wait()
        pltpu.async_copy(g_hbm.at[svs[p]], rws[p], ggs[p])

    @pl.loop(0, CHUNKS, step=6)
    def _(j):
        for half in range(2):
            for p in range(3):
                q = half * 3 + p
                pltpu.make_async_copy(g_hbm.at[svs[q]], rws[p],
                                      ggs[p]).wait()
                pltpu.async_copy(rws[p], acc.at[dvs[q]], sss[p], add=True)
            for p in range(3):
                q = half * 3 + p
                qn = (q + 3) % 6
                pltpu.make_async_copy(rws[p], acc.at[dvs[q]], sss[p]).wait()
                pltpu.make_async_copy(sidx(0), svs[qn], lis[qn]).wait()
                pltpu.make_async_copy(didx(0), dvs[qn], lis[qn]).wait()
                pltpu.async_copy(g_hbm.at[svs[qn]], rws[p], ggs[p])
                c2 = jnp.minimum(j + q + 6, CHUNKS - 1)
                pltpu.async_copy(sidx(c2), svs[q], lis[q])
                pltpu.async_copy(didx(c2), dvs[q], lis[q])

    for p in range(3):
        pltpu.make_async_copy(g_hbm.at[svs[p]], rws[p], ggs[p]).wait()
    for q in (3, 4, 5):
        pltpu.make_async_copy(sidx(0), svs[q], lis[q]).wait()
        pltpu.make_async_copy(didx(0), dvs[q], lis[q]).wait()

    plsc.subcore_barrier()
    _writeback(cid, sid, acc, p0_hbm, p1_hbm)


R = 1000
GRID = N // R


def _dis_block(d0, d1):
    deg = d0[:, 0:1] + d1[:, 0:1] + 1.0
    return lax.rsqrt(deg)


def _pre_body(x_ref, w_ref, d0_ref, d1_ref, h_ref, g_ref):
    dis = _dis_block(d0_ref[...], d1_ref[...])
    h = jnp.dot(x_ref[...], w_ref[...], preferred_element_type=jnp.float32)
    h_ref[...] = h
    g_ref[...] = h * dis


def _mid_body(p0_ref, p1_ref, hp_ref, d0_ref, d1_ref, w_ref, b_ref,
              h_ref, g_ref):
    dis = _dis_block(d0_ref[...], d1_ref[...])
    out = dis * (p0_ref[...] + p1_ref[...]) + (dis * dis) * hp_ref[...] \
        + b_ref[...]
    t = jnp.maximum(out, 0.0)
    h = jnp.dot(t, w_ref[...], preferred_element_type=jnp.float32)
    h_ref[...] = h
    g_ref[...] = h * dis


def _post_body(p0_ref, p1_ref, hp_ref, d0_ref, d1_ref, b_ref, o_ref):
    dis = _dis_block(d0_ref[...], d1_ref[...])
    o_ref[...] = dis * (p0_ref[...] + p1_ref[...]) \
        + (dis * dis) * hp_ref[...] + b_ref[...]


_row_spec = pl.BlockSpec((R, D), lambda i: (i, 0))
_deg_spec = pl.BlockSpec((R, D), lambda i: (i, 0))
_w_spec = pl.BlockSpec((D, D), lambda i: (0, 0))
_b_spec = pl.BlockSpec((1, D), lambda i: (0, 0))

_pre_call = pl.pallas_call(
    _pre_body,
    grid=(GRID,),
    in_specs=[_row_spec, _w_spec, _deg_spec, _deg_spec],
    out_specs=[_row_spec, _row_spec],
    out_shape=(_f32(N, D), _f32(N, D)),
)

_mid_call = pl.pallas_call(
    _mid_body,
    grid=(GRID,),
    in_specs=[_row_spec, _row_spec, _row_spec, _deg_spec, _deg_spec,
              _w_spec, _b_spec],
    out_specs=[_row_spec, _row_spec],
    out_shape=(_f32(N, D), _f32(N, D)),
)

_post_call = pl.pallas_call(
    _post_body,
    grid=(GRID,),
    in_specs=[_row_spec, _row_spec, _row_spec, _deg_spec, _deg_spec, _b_spec],
    out_specs=_row_spec,
    out_shape=_f32(N, D),
)


def kernel(x, edge_index, W1, b1, W2, b2, W3, b3):
    src = edge_index[0].astype(jnp.int32)
    dst = edge_index[1].astype(jnp.int32)

    pad_src = (jnp.arange(PAD, dtype=jnp.int32) * 97) % N
    pad_dst = N + (jnp.arange(PAD, dtype=jnp.int32) % 8)
    src = jnp.concatenate([src, pad_src])
    dst = jnp.concatenate([dst, pad_dst])

    zeros_row = jnp.zeros((ROWS_A, D), jnp.float32)
    ones_row = jnp.ones((C, D), jnp.float32)

    d0, d1 = _degree_kernel(dst, ones_row, zeros_row)

    h1, g1 = _pre_call(x, W1, d0, d1)
    p0, p1 = _propagate_kernel(g1, src, dst, zeros_row)
    h2, g2 = _mid_call(p0, p1, h1, d0, d1, W2, b1.reshape(1, D))
    p0, p1 = _propagate_kernel(g2, src, dst, zeros_row)
    h3, g3 = _mid_call(p0, p1, h2, d0, d1, W3, b2.reshape(1, D))
    p0, p1 = _propagate_kernel(g3, src, dst, zeros_row)
    out = _post_call(p0, p1, h3, d0, d1, b3.reshape(1, D))
    return out


# speedup vs baseline: 1.1060x; 1.1060x over previous
import dataclasses
import functools

import jax
import jax.numpy as jnp
from jax import lax
from jax.experimental import pallas as pl
from jax.experimental.pallas import tpu as pltpu
from jax.experimental.pallas import tpu_sc as plsc

N = 10000
D = 128
E = 320000

NC = 2
NS = 16
NW = NC * NS

C = 120
CHUNKS = 84
EPW = C * CHUNKS
E2 = EPW * NW
PAD = E2 - E
NA = N + 8

ROWS_A = 632
ROWS_B = N - 15 * ROWS_A

_mesh = plsc.VectorSubcoreMesh(core_axis_name="c", subcore_axis_name="s")
_cp = pltpu.CompilerParams()
if "needs_layout_passes" in pltpu.CompilerParams.__dataclass_fields__:
    _cp = dataclasses.replace(_cp, needs_layout_passes=False)


def _f32(*shape):
    return jax.ShapeDtypeStruct(shape, jnp.float32)


def _zero_acc(sid, zeros_hbm, acc):
    @pl.when(sid < NS - 1)
    def _():
        pltpu.sync_copy(zeros_hbm, acc.at[pl.ds(sid * ROWS_A, ROWS_A)])

    @pl.when(sid == NS - 1)
    def _():
        pltpu.sync_copy(zeros_hbm.at[pl.ds(0, ROWS_B)],
                        acc.at[pl.ds(sid * ROWS_A, ROWS_B)])


def _writeback(cid, sid, acc, p0_hbm, p1_hbm):
    @pl.when(sid < NS - 1)
    def _():
        @pl.when(cid == 0)
        def _():
            pltpu.sync_copy(acc.at[pl.ds(sid * ROWS_A, ROWS_A)],
                            p0_hbm.at[pl.ds(sid * ROWS_A, ROWS_A)])

        @pl.when(cid == 1)
        def _():
            pltpu.sync_copy(acc.at[pl.ds(sid * ROWS_A, ROWS_A)],
                            p1_hbm.at[pl.ds(sid * ROWS_A, ROWS_A)])

    @pl.when(sid == NS - 1)
    def _():
        @pl.when(cid == 0)
        def _():
            pltpu.sync_copy(acc.at[pl.ds(sid * ROWS_A, ROWS_B)],
                            p0_hbm.at[pl.ds(sid * ROWS_A, ROWS_B)])

        @pl.when(cid == 1)
        def _():
            pltpu.sync_copy(acc.at[pl.ds(sid * ROWS_A, ROWS_B)],
                            p1_hbm.at[pl.ds(sid * ROWS_A, ROWS_B)])


NH = 10240
ROWS_C = 640
LAST_C = N - 15 * ROWS_C


@functools.partial(
    pl.kernel,
    out_type=(jax.ShapeDtypeStruct((N,), jnp.int32),
              jax.ShapeDtypeStruct((N,), jnp.int32)),
    mesh=_mesh,
    compiler_params=_cp,
    scratch_types=[
        pltpu.VMEM((EPW,), jnp.int32),
        pltpu.VMEM((NH,), jnp.int32),
        pltpu.VMEM((16, ROWS_C), jnp.int32),
        pltpu.VMEM((ROWS_C,), jnp.int32),
        pltpu.VMEM_SHARED((16, NH), jnp.int32),
        pltpu.SemaphoreType.DMA,
    ],
)
def _degree_kernel(dst_hbm, d0_hbm, d1_hbm, dstv, hist, stage, red, hists,
                   sem):
    cid = lax.axis_index("c")
    sid = lax.axis_index("s")
    wid = cid * NS + sid

    pltpu.async_copy(dst_hbm.at[pl.ds(wid * EPW, EPW)], dstv, sem)

    @pl.loop(0, NH, step=16)
    def _(i):
        hist[pl.ds(i, 16)] = jnp.zeros((16,), jnp.int32)

    pltpu.make_async_copy(dst_hbm.at[pl.ds(wid * EPW, EPW)], dstv, sem).wait()

    @pl.loop(0, EPW, step=16)
    def _(e):
        v = dstv[pl.ds(e, 16)]
        cnt, m = plsc.scan_count(v)
        plsc.addupdate_scatter(hist, [v], cnt, mask=m)

    pltpu.sync_copy(hist, hists.at[sid])
    plsc.subcore_barrier()

    base = sid * ROWS_C
    pltpu.sync_copy(hists.at[:, pl.ds(base, ROWS_C)], stage)

    @pl.loop(0, ROWS_C, step=16)
    def _(i):
        acc = stage[0, pl.ds(i, 16)]
        for t in range(1, 16):
            acc = acc + stage[t, pl.ds(i, 16)]
        red[pl.ds(i, 16)] = acc

    @pl.when(sid < NS - 1)
    def _():
        @pl.when(cid == 0)
        def _():
            pltpu.sync_copy(red, d0_hbm.at[pl.ds(base, ROWS_C)])

        @pl.when(cid == 1)
        def _():
            pltpu.sync_copy(red, d1_hbm.at[pl.ds(base, ROWS_C)])

    @pl.when(sid == NS - 1)
    def _():
        @pl.when(cid == 0)
        def _():
            pltpu.sync_copy(red.at[pl.ds(0, LAST_C)],
                            d0_hbm.at[pl.ds(base, LAST_C)])

        @pl.when(cid == 1)
        def _():
            pltpu.sync_copy(red.at[pl.ds(0, LAST_C)],
                            d1_hbm.at[pl.ds(base, LAST_C)])


@functools.partial(
    pl.kernel,
    out_type=(_f32(N, D), _f32(N, D)),
    mesh=_mesh,
    scratch_types=[
        [pltpu.VMEM((C,), jnp.int32)] * 6,
        [pltpu.VMEM((C,), jnp.int32)] * 6,
        [pltpu.VMEM((C, D), jnp.float32)] * 3,
        pltpu.VMEM_SHARED((NA, D), jnp.float32),
        [pltpu.SemaphoreType.DMA] * 6,
        [pltpu.SemaphoreType.DMA] * 3,
        [pltpu.SemaphoreType.DMA] * 3,
    ],
)
def _propagate_kernel(g_hbm, src_hbm, dst_hbm, zeros_hbm, p0_hbm, p1_hbm,
                      svs, dvs, rws, acc, lis, ggs, sss):
    cid = lax.axis_index("c")
    sid = lax.axis_index("s")
    wid = cid * NS + sid
    base = wid * EPW

    def sidx(c):
        return src_hbm.at[pl.ds(base + c * C, C)]

    def didx(c):
        return dst_hbm.at[pl.ds(base + c * C, C)]

    _zero_acc(sid, zeros_hbm, acc)

    for q in range(6):
        pltpu.async_copy(sidx(q), svs[q], lis[q])
        pltpu.async_copy(didx(q), dvs[q], lis[q])

    plsc.subcore_barrier()

    for p in range(3):
        pltpu.make_async_copy(sidx(p), svs[p], lis[p]).wait()
        pltpu.make_async_copy(didx(p), dvs[p], lis[p]).wait()
        pltpu.async_copy(g_hbm.at[svs[p]], rws[p], ggs[p])

    @pl.loop(0, CHUNKS, step=6)
    def _(j):
        for half in range(2):
            for p in range(3):
                q = half * 3 + p
                pltpu.make_async_copy(g_hbm.at[svs[q]], rws[p],
                                      ggs[p]).wait()
                pltpu.async_copy(rws[p], acc.at[dvs[q]], sss[p], add=True)
            for p in range(3):
                q = half * 3 + p
                qn = (q + 3) % 6
                pltpu.make_async_copy(rws[p], acc.at[dvs[q]], sss[p]).wait()
                pltpu.make_async_copy(sidx(0), svs[qn], lis[qn]).wait()
                pltpu.make_async_copy(didx(0), dvs[qn], lis[qn]).wait()
                pltpu.async_copy(g_hbm.at[svs[qn]], rws[p], ggs[p])
                c2 = jnp.minimum(j + q + 6, CHUNKS - 1)
                pltpu.async_copy(sidx(c2), svs[q], lis[q])
                pltpu.async_copy(didx(c2), dvs[q], lis[q])

    for p in range(3):
        pltpu.make_async_copy(g_hbm.at[svs[p]], rws[p], ggs[p]).wait()
    for q in (3, 4, 5):
        pltpu.make_async_copy(sidx(0), svs[q], lis[q]).wait()
        pltpu.make_async_copy(didx(0), dvs[q], lis[q]).wait()

    plsc.subcore_barrier()
    _writeback(cid, sid, acc, p0_hbm, p1_hbm)


R = 1000
GRID = N // R


def _dis_block(d0, d1):
    deg = (d0 + d1).astype(jnp.float32) + 1.0
    return lax.rsqrt(deg)


def _pre_body(x_ref, w_ref, d0_ref, d1_ref, h_ref, g_ref):
    dis = _dis_block(d0_ref[...], d1_ref[...])
    h = jnp.dot(x_ref[...], w_ref[...], preferred_element_type=jnp.float32)
    h_ref[...] = h
    g_ref[...] = h * dis


def _mid_body(p0_ref, p1_ref, hp_ref, d0_ref, d1_ref, w_ref, b_ref,
              h_ref, g_ref):
    dis = _dis_block(d0_ref[...], d1_ref[...])
    out = dis * (p0_ref[...] + p1_ref[...]) + (dis * dis) * hp_ref[...] \
        + b_ref[...]
    t = jnp.maximum(out, 0.0)
    h = jnp.dot(t, w_ref[...], preferred_element_type=jnp.float32)
    h_ref[...] = h
    g_ref[...] = h * dis


def _post_body(p0_ref, p1_ref, hp_ref, d0_ref, d1_ref, b_ref, o_ref):
    dis = _dis_block(d0_ref[...], d1_ref[...])
    o_ref[...] = dis * (p0_ref[...] + p1_ref[...]) \
        + (dis * dis) * hp_ref[...] + b_ref[...]


_row_spec = pl.BlockSpec((R, D), lambda i: (i, 0))
_deg_spec = pl.BlockSpec((R, 1), lambda i: (i, 0))
_w_spec = pl.BlockSpec((D, D), lambda i: (0, 0))
_b_spec = pl.BlockSpec((1, D), lambda i: (0, 0))

_pre_call = pl.pallas_call(
    _pre_body,
    grid=(GRID,),
    in_specs=[_row_spec, _w_spec, _deg_spec, _deg_spec],
    out_specs=[_row_spec, _row_spec],
    out_shape=(_f32(N, D), _f32(N, D)),
)

_mid_call = pl.pallas_call(
    _mid_body,
    grid=(GRID,),
    in_specs=[_row_spec, _row_spec, _row_spec, _deg_spec, _deg_spec,
              _w_spec, _b_spec],
    out_specs=[_row_spec, _row_spec],
    out_shape=(_f32(N, D), _f32(N, D)),
)

_post_call = pl.pallas_call(
    _post_body,
    grid=(GRID,),
    in_specs=[_row_spec, _row_spec, _row_spec, _deg_spec, _deg_spec, _b_spec],
    out_specs=_row_spec,
    out_shape=_f32(N, D),
)


def kernel(x, edge_index, W1, b1, W2, b2, W3, b3):
    src = edge_index[0].astype(jnp.int32)
    dst = edge_index[1].astype(jnp.int32)

    pad_src = (jnp.arange(PAD, dtype=jnp.int32) * 97) % N
    pad_dst = N + (jnp.arange(PAD, dtype=jnp.int32) % 8)
    src = jnp.concatenate([src, pad_src])
    dst = jnp.concatenate([dst, pad_dst])

    zeros_row = jnp.zeros((ROWS_A, D), jnp.float32)

    d0, d1 = _degree_kernel(dst)
    d0 = d0.reshape(N, 1)
    d1 = d1.reshape(N, 1)

    h1, g1 = _pre_call(x, W1, d0, d1)
    p0, p1 = _propagate_kernel(g1, src, dst, zeros_row)
    h2, g2 = _mid_call(p0, p1, h1, d0, d1, W2, b1.reshape(1, D))
    p0, p1 = _propagate_kernel(g2, src, dst, zeros_row)
    h3, g3 = _mid_call(p0, p1, h2, d0, d1, W3, b2.reshape(1, D))
    p0, p1 = _propagate_kernel(g3, src, dst, zeros_row)
    out = _post_call(p0, p1, h3, d0, d1, b3.reshape(1, D))
    return out


# speedup vs baseline: 1.1451x; 1.0354x over previous
import dataclasses
import functools

import jax
import jax.numpy as jnp
from jax import lax
from jax.experimental import pallas as pl
from jax.experimental.pallas import tpu as pltpu
from jax.experimental.pallas import tpu_sc as plsc

N = 10000
D = 128
E = 320000

NC = 2
NS = 16
NW = NC * NS

C = 120
CHUNKS = 84
EPW = C * CHUNKS
E2 = EPW * NW
PAD = E2 - E
NA = N + 8

ROWS_A = 632
ROWS_B = N - 15 * ROWS_A

_mesh = plsc.VectorSubcoreMesh(core_axis_name="c", subcore_axis_name="s")
_cp = pltpu.CompilerParams()
if "needs_layout_passes" in pltpu.CompilerParams.__dataclass_fields__:
    _cp = dataclasses.replace(_cp, needs_layout_passes=False)


def _f32(*shape):
    return jax.ShapeDtypeStruct(shape, jnp.float32)


def _zero_acc(sid, zeros_hbm, acc):
    @pl.when(sid < NS - 1)
    def _():
        pltpu.sync_copy(zeros_hbm, acc.at[pl.ds(sid * ROWS_A, ROWS_A)])

    @pl.when(sid == NS - 1)
    def _():
        pltpu.sync_copy(zeros_hbm.at[pl.ds(0, ROWS_B)],
                        acc.at[pl.ds(sid * ROWS_A, ROWS_B)])


def _writeback(cid, sid, acc, p0_hbm, p1_hbm):
    @pl.when(sid < NS - 1)
    def _():
        @pl.when(cid == 0)
        def _():
            pltpu.sync_copy(acc.at[pl.ds(sid * ROWS_A, ROWS_A)],
                            p0_hbm.at[pl.ds(sid * ROWS_A, ROWS_A)])

        @pl.when(cid == 1)
        def _():
            pltpu.sync_copy(acc.at[pl.ds(sid * ROWS_A, ROWS_A)],
                            p1_hbm.at[pl.ds(sid * ROWS_A, ROWS_A)])

    @pl.when(sid == NS - 1)
    def _():
        @pl.when(cid == 0)
        def _():
            pltpu.sync_copy(acc.at[pl.ds(sid * ROWS_A, ROWS_B)],
                            p0_hbm.at[pl.ds(sid * ROWS_A, ROWS_B)])

        @pl.when(cid == 1)
        def _():
            pltpu.sync_copy(acc.at[pl.ds(sid * ROWS_A, ROWS_B)],
                            p1_hbm.at[pl.ds(sid * ROWS_A, ROWS_B)])


NH = 10240
ROWS_C = 640
LAST_C = N - 15 * ROWS_C


@functools.partial(
    pl.kernel,
    out_type=(jax.ShapeDtypeStruct((N,), jnp.int32),
              jax.ShapeDtypeStruct((N,), jnp.int32)),
    mesh=_mesh,
    compiler_params=_cp,
    scratch_types=[
        pltpu.VMEM((EPW,), jnp.int32),
        pltpu.VMEM((NH,), jnp.int32),
        pltpu.VMEM((16, ROWS_C), jnp.int32),
        pltpu.VMEM((ROWS_C,), jnp.int32),
        pltpu.VMEM_SHARED((16, NH), jnp.int32),
        pltpu.SemaphoreType.DMA,
    ],
)
def _degree_kernel(dst_hbm, d0_hbm, d1_hbm, dstv, hist, stage, red, hists,
                   sem):
    cid = lax.axis_index("c")
    sid = lax.axis_index("s")
    wid = cid * NS + sid

    pltpu.async_copy(dst_hbm.at[pl.ds(wid * EPW, EPW)], dstv, sem)

    @pl.loop(0, NH, step=16)
    def _(i):
        hist[pl.ds(i, 16)] = jnp.zeros((16,), jnp.int32)

    pltpu.make_async_copy(dst_hbm.at[pl.ds(wid * EPW, EPW)], dstv, sem).wait()

    @pl.loop(0, EPW, step=16)
    def _(e):
        v = dstv[pl.ds(e, 16)]
        cnt, m = plsc.scan_count(v)
        plsc.addupdate_scatter(hist, [v], cnt, mask=m)

    pltpu.sync_copy(hist, hists.at[sid])
    plsc.subcore_barrier()

    base = sid * ROWS_C
    pltpu.sync_copy(hists.at[:, pl.ds(base, ROWS_C)], stage)

    @pl.loop(0, ROWS_C, step=16)
    def _(i):
        acc = stage[0, pl.ds(i, 16)]
        for t in range(1, 16):
            acc = acc + stage[t, pl.ds(i, 16)]
        red[pl.ds(i, 16)] = acc

    @pl.when(sid < NS - 1)
    def _():
        @pl.when(cid == 0)
        def _():
            pltpu.sync_copy(red, d0_hbm.at[pl.ds(base, ROWS_C)])

        @pl.when(cid == 1)
        def _():
            pltpu.sync_copy(red, d1_hbm.at[pl.ds(base, ROWS_C)])

    @pl.when(sid == NS - 1)
    def _():
        @pl.when(cid == 0)
        def _():
            pltpu.sync_copy(red.at[pl.ds(0, LAST_C)],
                            d0_hbm.at[pl.ds(base, LAST_C)])

        @pl.when(cid == 1)
        def _():
            pltpu.sync_copy(red.at[pl.ds(0, LAST_C)],
                            d1_hbm.at[pl.ds(base, LAST_C)])


@functools.partial(
    pl.kernel,
    out_type=(_f32(N, D), _f32(N, D)),
    mesh=_mesh,
    scratch_types=[
        [pltpu.VMEM((C,), jnp.int32)] * 3,
        [pltpu.VMEM((C,), jnp.int32)] * 3,
        [pltpu.VMEM((C, D), jnp.float32)] * 3,
        pltpu.VMEM_SHARED((NA, D), jnp.float32),
        [pltpu.SemaphoreType.DMA] * 3,
        [pltpu.SemaphoreType.DMA] * 3,
        [pltpu.SemaphoreType.DMA] * 3,
    ],
)
def _propagate_kernel(g_hbm, src_hbm, dst_hbm, p0_hbm, p1_hbm,
                      svs, dvs, rws, acc, lis, ggs, sss):
    cid = lax.axis_index("c")
    sid = lax.axis_index("s")
    wid = cid * NS + sid
    base = wid * EPW

    def sidx(c):
        return src_hbm.at[pl.ds(base + c * C, C)]

    def didx(c):
        return dst_hbm.at[pl.ds(base + c * C, C)]

    @pl.loop(0, C)
    def _(i):
        for j in range(D // 16):
            rws[0][i, pl.ds(j * 16, 16)] = jnp.zeros((16,), jnp.float32)

    zbase = sid * ROWS_A

    @pl.when(sid < NS - 1)
    def _():
        for k in range(5):
            pltpu.sync_copy(rws[0], acc.at[pl.ds(zbase + k * C, C)])
        pltpu.sync_copy(rws[0].at[pl.ds(0, ROWS_A - 5 * C)],
                        acc.at[pl.ds(zbase + 5 * C, ROWS_A - 5 * C)])

    @pl.when(sid == NS - 1)
    def _():
        for k in range(4):
            pltpu.sync_copy(rws[0], acc.at[pl.ds(zbase + k * C, C)])
        pltpu.sync_copy(rws[0].at[pl.ds(0, ROWS_B - 4 * C)],
                        acc.at[pl.ds(zbase + 4 * C, ROWS_B - 4 * C)])

    for p in range(3):
        pltpu.async_copy(sidx(p), svs[p], lis[p])
        pltpu.async_copy(didx(p), dvs[p], lis[p])

    plsc.subcore_barrier()

    for p in range(3):
        pltpu.make_async_copy(sidx(p), svs[p], lis[p]).wait()
        pltpu.make_async_copy(didx(p), dvs[p], lis[p]).wait()
        pltpu.async_copy(g_hbm.at[svs[p]], rws[p], ggs[p])

    @pl.loop(0, CHUNKS, step=3)
    def _(j):
        for p in range(3):
            pltpu.make_async_copy(g_hbm.at[svs[p]], rws[p], ggs[p]).wait()
            pltpu.async_copy(rws[p], acc.at[dvs[p]], sss[p], add=True)
        for p in range(3):
            c2 = jnp.minimum(j + 3 + p, CHUNKS - 1)
            pltpu.make_async_copy(rws[p], acc.at[dvs[p]], sss[p]).wait()
            pltpu.async_copy(sidx(c2), svs[p], lis[p])
            pltpu.async_copy(didx(c2), dvs[p], lis[p])
            pltpu.make_async_copy(sidx(c2), svs[p], lis[p]).wait()
            pltpu.make_async_copy(didx(c2), dvs[p], lis[p]).wait()
            pltpu.async_copy(g_hbm.at[svs[p]], rws[p], ggs[p])

    for p in range(3):
        pltpu.make_async_copy(g_hbm.at[svs[p]], rws[p], ggs[p]).wait()

    plsc.subcore_barrier()
    _writeback(cid, sid, acc, p0_hbm, p1_hbm)


R = 1000
GRID = N // R


def _dis_block(d0, d1):
    deg = (d0 + d1).astype(jnp.float32) + 1.0
    return lax.rsqrt(deg)


def _pre_body(x_ref, w_ref, d0_ref, d1_ref, h_ref, g_ref):
    dis = _dis_block(d0_ref[...], d1_ref[...])
    h = jnp.dot(x_ref[...], w_ref[...], preferred_element_type=jnp.float32)
    h_ref[...] = h
    g_ref[...] = h * dis


def _mid_body(p0_ref, p1_ref, hp_ref, d0_ref, d1_ref, w_ref, b_ref,
              h_ref, g_ref):
    dis = _dis_block(d0_ref[...], d1_ref[...])
    out = dis * (p0_ref[...] + p1_ref[...]) + (dis * dis) * hp_ref[...] \
        + b_ref[...]
    t = jnp.maximum(out, 0.0)
    h = jnp.dot(t, w_ref[...], preferred_element_type=jnp.float32)
    h_ref[...] = h
    g_ref[...] = h * dis


def _post_body(p0_ref, p1_ref, hp_ref, d0_ref, d1_ref, b_ref, o_ref):
    dis = _dis_block(d0_ref[...], d1_ref[...])
    o_ref[...] = dis * (p0_ref[...] + p1_ref[...]) \
        + (dis * dis) * hp_ref[...] + b_ref[...]


_row_spec = pl.BlockSpec((R, D), lambda i: (i, 0))
_deg_spec = pl.BlockSpec((R, 1), lambda i: (i, 0))
_w_spec = pl.BlockSpec((D, D), lambda i: (0, 0))
_b_spec = pl.BlockSpec((1, D), lambda i: (0, 0))

_pre_call = pl.pallas_call(
    _pre_body,
    grid=(GRID,),
    in_specs=[_row_spec, _w_spec, _deg_spec, _deg_spec],
    out_specs=[_row_spec, _row_spec],
    out_shape=(_f32(N, D), _f32(N, D)),
)

_mid_call = pl.pallas_call(
    _mid_body,
    grid=(GRID,),
    in_specs=[_row_spec, _row_spec, _row_spec, _deg_spec, _deg_spec,
              _w_spec, _b_spec],
    out_specs=[_row_spec, _row_spec],
    out_shape=(_f32(N, D), _f32(N, D)),
)

_post_call = pl.pallas_call(
    _post_body,
    grid=(GRID,),
    in_specs=[_row_spec, _row_spec, _row_spec, _deg_spec, _deg_spec, _b_spec],
    out_specs=_row_spec,
    out_shape=_f32(N, D),
)


def kernel(x, edge_index, W1, b1, W2, b2, W3, b3):
    src = edge_index[0].astype(jnp.int32)
    dst = edge_index[1].astype(jnp.int32)

    pad_src = (jnp.arange(PAD, dtype=jnp.int32) * 97) % N
    pad_dst = N + (jnp.arange(PAD, dtype=jnp.int32) % 8)
    src = jnp.concatenate([src, pad_src])
    dst = jnp.concatenate([dst, pad_dst])


    d0, d1 = _degree_kernel(dst)
    d0 = d0.reshape(N, 1)
    d1 = d1.reshape(N, 1)

    h1, g1 = _pre_call(x, W1, d0, d1)
    p0, p1 = _propagate_kernel(g1, src, dst)
    h2, g2 = _mid_call(p0, p1, h1, d0, d1, W2, b1.reshape(1, D))
    p0, p1 = _propagate_kernel(g2, src, dst)
    h3, g3 = _mid_call(p0, p1, h2, d0, d1, W3, b2.reshape(1, D))
    p0, p1 = _propagate_kernel(g3, src, dst)
    out = _post_call(p0, p1, h3, d0, d1, b3.reshape(1, D))
    return out
